# baseline (device time: 56351 ns/iter reference)
import jax
import jax.numpy as jnp
from jax import lax
from jax.experimental import pallas as pl
from jax.experimental.pallas import tpu as pltpu

N_DEV = 8
B_PER = 2
SQ = 256
H_PER = 4
DH = 64
DM = 512
DQ = H_PER * DH

R_HOPS = 4
L_HOPS = 3


def kernel(x, Wq, K_ext, V_ext, Wo):
    p = lax.axis_index("i")

    k_l = lax.dynamic_slice(K_ext, (2 * p, 0, 0, 0), (B_PER, SQ, 32, DH))
    v_l = lax.dynamic_slice(V_ext, (2 * p, 0, 0, 0), (B_PER, SQ, 32, DH))
    k16 = jnp.transpose(k_l, (2, 0, 1, 3)).astype(jnp.bfloat16)
    v16 = jnp.transpose(v_l, (2, 0, 1, 3)).astype(jnp.bfloat16)
    x16 = x.astype(jnp.bfloat16)
    wq16 = Wq.astype(jnp.bfloat16)
    wo16 = Wo.astype(jnp.bfloat16)

    def body(x_ref, wq_ref, k_ref, v_ref, wo_ref, out_ref,
             wq_all, wo_all,
             r_send_sems, r_recv_sems, l_send_sems, l_recv_sems):
        my = lax.axis_index("i")
        left = lax.rem(my - 1 + N_DEV, N_DEV)
        right = lax.rem(my + 1, N_DEV)

        xb = x_ref[...].reshape(B_PER * SQ, DM)
        NB = B_PER * SQ // 64

        def add_chunk(c, acc):
            Q = jnp.dot(xb, wq_all[c],
                        preferred_element_type=jnp.float32)
            Qb = Q.astype(jnp.bfloat16)
            for h in range(H_PER):
                kh = k_ref[H_PER * c + h].reshape(NB, 64, DH)
                vh = v_ref[H_PER * c + h].reshape(NB, 64, DH)
                qh = Qb[:, h * DH:(h + 1) * DH].reshape(NB, 64, DH)
                scores = lax.dot_general(
                    qh, kh,
                    dimension_numbers=(((2,), (2,)), ((0,), (0,))),
                    preferred_element_type=jnp.float32,
                ) * 0.125
                m = jnp.max(scores, axis=-1, keepdims=True)
                w = jnp.exp(scores - m)
                w = (w / jnp.sum(w, axis=-1, keepdims=True)
                     ).astype(jnp.bfloat16)
                ctx = lax.dot_general(
                    w, vh,
                    dimension_numbers=(((2,), (1,)), ((0,), (0,))),
                    preferred_element_type=jnp.float32,
                ).astype(jnp.bfloat16)
                ph = jnp.dot(ctx.reshape(B_PER * SQ, DH),
                             wo_all[c, h * DH:(h + 1) * DH, :],
                             preferred_element_type=jnp.float32)
                acc = acc + ph
            return acc

        barrier = pltpu.get_barrier_semaphore()
        pl.semaphore_signal(barrier, inc=1, device_id=(left,),
                            device_id_type=pl.DeviceIdType.MESH)
        pl.semaphore_signal(barrier, inc=1, device_id=(right,),
                            device_id_type=pl.DeviceIdType.MESH)
        pl.semaphore_wait(barrier, 2)

        sends = []

        def send_pair(chunk, hop, to, send_sems, recv_sems):
            for buf, s in ((wq_all, 0), (wo_all, 1)):
                rdma = pltpu.make_async_remote_copy(
                    src_ref=buf.at[chunk],
                    dst_ref=buf.at[chunk],
                    send_sem=send_sems.at[2 * hop + s],
                    recv_sem=recv_sems.at[2 * hop + s],
                    device_id=(to,),
                    device_id_type=pl.DeviceIdType.MESH,
                )
                rdma.start()
                sends.append(rdma)
            return sends[-2:]

        def wait_pair(pair):
            pair[0].wait_recv()
            pair[1].wait_recv()

        wq_all[my] = wq_ref[...]
        wo_all[my] = wo_ref[...]
        r_hops = [send_pair(my, 0, right, r_send_sems, r_recv_sems)]
        l_hops = [send_pair(my, 0, left, l_send_sems, l_recv_sems)]
        acc = add_chunk(my, jnp.zeros((B_PER * SQ, DM), jnp.float32))

        for s in range(R_HOPS):
            cr = lax.rem(my - 1 - s + N_DEV, N_DEV)
            wait_pair(r_hops[s])
            if s + 1 < R_HOPS:
                r_hops.append(
                    send_pair(cr, s + 1, right, r_send_sems, r_recv_sems))
            if s < L_HOPS:
                cl = lax.rem(my + 1 + s, N_DEV)
                wait_pair(l_hops[s])
                if s + 1 < L_HOPS:
                    l_hops.append(
                        send_pair(cl, s + 1, left, l_send_sems, l_recv_sems))
                acc = add_chunk(cr, acc)
                acc = add_chunk(cl, acc)
            else:
                acc = add_chunk(cr, acc)

        out_ref[...] = acc.reshape(B_PER, SQ, DM)

        for rdma in sends:
            rdma.wait_send()

    grid_spec = pltpu.PrefetchScalarGridSpec(
        num_scalar_prefetch=0,
        in_specs=[
            pl.BlockSpec(memory_space=pltpu.VMEM),
            pl.BlockSpec(memory_space=pltpu.VMEM),
            pl.BlockSpec(memory_space=pltpu.VMEM),
            pl.BlockSpec(memory_space=pltpu.VMEM),
            pl.BlockSpec(memory_space=pltpu.VMEM),
        ],
        out_specs=pl.BlockSpec(memory_space=pltpu.VMEM),
        scratch_shapes=[
            pltpu.VMEM((N_DEV, DM, DQ), jnp.bfloat16),
            pltpu.VMEM((N_DEV, DQ, DM), jnp.bfloat16),
            pltpu.SemaphoreType.DMA((2 * R_HOPS,)),
            pltpu.SemaphoreType.DMA((2 * R_HOPS,)),
            pltpu.SemaphoreType.DMA((2 * L_HOPS,)),
            pltpu.SemaphoreType.DMA((2 * L_HOPS,)),
        ],
    )

    return pl.pallas_call(
        body,
        out_shape=jax.ShapeDtypeStruct((B_PER, SQ, DM), jnp.float32),
        grid_spec=grid_spec,
        compiler_params=pltpu.CompilerParams(
            collective_id=0,
            vmem_limit_bytes=100 * 1024 * 1024,
        ),
    )(x16, wq16, k16, v16, wo16)


# device time: 50984 ns/iter; 1.1053x vs baseline; 1.1053x over previous
import jax
import jax.numpy as jnp
from jax import lax
from jax.experimental import pallas as pl
from jax.experimental.pallas import tpu as pltpu

N_DEV = 8
B_PER = 2
SQ = 256
H_PER = 4
DH = 64
DM = 512
DQ = H_PER * DH

N_XFERS = 7


def kernel(x, Wq, K_ext, V_ext, Wo):
    p = lax.axis_index("i")

    k_l = lax.dynamic_slice(K_ext, (2 * p, 0, 0, 0), (B_PER, SQ, 32, DH))
    v_l = lax.dynamic_slice(V_ext, (2 * p, 0, 0, 0), (B_PER, SQ, 32, DH))
    k16 = jnp.transpose(k_l, (2, 0, 1, 3)).astype(jnp.bfloat16)
    v16 = jnp.transpose(v_l, (2, 0, 1, 3)).astype(jnp.bfloat16)
    x16 = x.astype(jnp.bfloat16)
    wq16 = Wq.astype(jnp.bfloat16)
    wo16 = Wo.astype(jnp.bfloat16)

    def body(x_ref, wq_ref, k_ref, v_ref, wo_ref, out_ref,
             wq_all, wo_all, send_sems, recv_sems):
        my = lax.axis_index("i")

        def pi(t):
            t = lax.rem(t + 32, N_DEV)
            return jnp.where(t < 4, t, 11 - t)

        pos = pi(my)
        even = lax.rem(pos, 2) == 0
        right = pi(pos + 1)
        left = pi(pos - 1)
        partner = pi(pos + jnp.where(even, 3, -3))

        xb = x_ref[...].reshape(B_PER * SQ, DM)
        NB = B_PER * SQ // 64

        def add_chunk(c, acc):
            Q = jnp.dot(xb, wq_all[c],
                        preferred_element_type=jnp.float32)
            Qb = Q.astype(jnp.bfloat16)
            for h in range(H_PER):
                kh = k_ref[H_PER * c + h].reshape(NB, 64, DH)
                vh = v_ref[H_PER * c + h].reshape(NB, 64, DH)
                qh = Qb[:, h * DH:(h + 1) * DH].reshape(NB, 64, DH)
                scores = lax.dot_general(
                    qh, kh,
                    dimension_numbers=(((2,), (2,)), ((0,), (0,))),
                    preferred_element_type=jnp.float32,
                ) * 0.125
                m = jnp.max(scores, axis=-1, keepdims=True)
                w = jnp.exp(scores - m)
                w = (w / jnp.sum(w, axis=-1, keepdims=True)
                     ).astype(jnp.bfloat16)
                ctx = lax.dot_general(
                    w, vh,
                    dimension_numbers=(((2,), (1,)), ((0,), (0,))),
                    preferred_element_type=jnp.float32,
                ).astype(jnp.bfloat16)
                ph = jnp.dot(ctx.reshape(B_PER * SQ, DH),
                             wo_all[c, h * DH:(h + 1) * DH, :],
                             preferred_element_type=jnp.float32)
                acc = acc + ph
            return acc

        barrier = pltpu.get_barrier_semaphore()
        for nb in (left, right, partner):
            pl.semaphore_signal(barrier, inc=1, device_id=(nb,),
                                device_id_type=pl.DeviceIdType.MESH)
        pl.semaphore_wait(barrier, 3)

        sends = []

        def send_pair(chunk, slot, to):
            pair = []
            for buf, s in ((wq_all, 0), (wo_all, 1)):
                rdma = pltpu.make_async_remote_copy(
                    src_ref=buf.at[chunk],
                    dst_ref=buf.at[chunk],
                    send_sem=send_sems.at[2 * slot + s],
                    recv_sem=recv_sems.at[2 * slot + s],
                    device_id=(to,),
                    device_id_type=pl.DeviceIdType.MESH,
                )
                rdma.start()
                sends.append(rdma)
                pair.append(rdma)
            return pair

        def wait_pair(pair):
            pair[0].wait_recv()
            pair[1].wait_recv()

        zero = jnp.zeros((B_PER * SQ, DM), jnp.float32)

        wq_all[my] = wq_ref[...]
        wo_all[my] = wo_ref[...]
        s0r = send_pair(my, 0, right)
        s0l = send_pair(my, 1, left)
        s0p = send_pair(my, 2, partner)
        acc = add_chunk(my, zero)
        wait_pair(s0r)
        wait_pair(s0l)
        wait_pair(s0p)

        c_m1 = pi(pos - 1)
        c_p1 = pi(pos + 1)
        c_pn = pi(pos + jnp.where(even, 3, -3))
        s1r = send_pair(c_m1, 3, right)
        s1l = send_pair(c_p1, 4, left)
        s1p = send_pair(jnp.where(even, c_m1, c_p1), 5, partner)
        acc = add_chunk(c_m1, acc)
        acc = add_chunk(c_p1, acc)
        acc = add_chunk(c_pn, acc)
        wait_pair(s1r)
        wait_pair(s1l)
        wait_pair(s1p)

        c_m2 = pi(pos - 2)
        c_p2 = pi(pos + 2)
        s2 = send_pair(jnp.where(even, c_p2, c_m2), 6,
                       jnp.where(even, left, right))
        acc = add_chunk(c_m2, acc)
        acc = add_chunk(c_p2, acc)
        acc = add_chunk(pi(pos + 4), acc)
        wait_pair(s2)
        acc = add_chunk(pi(pos + jnp.where(even, -3, 3)), acc)

        out_ref[...] = acc.reshape(B_PER, SQ, DM)

        for rdma in sends:
            rdma.wait_send()

    grid_spec = pltpu.PrefetchScalarGridSpec(
        num_scalar_prefetch=0,
        in_specs=[
            pl.BlockSpec(memory_space=pltpu.VMEM),
            pl.BlockSpec(memory_space=pltpu.VMEM),
            pl.BlockSpec(memory_space=pltpu.VMEM),
            pl.BlockSpec(memory_space=pltpu.VMEM),
            pl.BlockSpec(memory_space=pltpu.VMEM),
        ],
        out_specs=pl.BlockSpec(memory_space=pltpu.VMEM),
        scratch_shapes=[
            pltpu.VMEM((N_DEV, DM, DQ), jnp.bfloat16),
            pltpu.VMEM((N_DEV, DQ, DM), jnp.bfloat16),
            pltpu.SemaphoreType.DMA((2 * N_XFERS,)),
            pltpu.SemaphoreType.DMA((2 * N_XFERS,)),
        ],
    )

    return pl.pallas_call(
        body,
        out_shape=jax.ShapeDtypeStruct((B_PER, SQ, DM), jnp.float32),
        grid_spec=grid_spec,
        compiler_params=pltpu.CompilerParams(
            collective_id=0,
            vmem_limit_bytes=100 * 1024 * 1024,
        ),
    )(x16, wq16, k16, v16, wo16)
